# baseline (speedup 1.0000x reference)
import functools

import numpy as np

import jax
import jax.numpy as jnp
from jax import lax
from jax.experimental import pallas as pl
from jax.experimental.pallas import tpu as pltpu
from jax.experimental.pallas import tpu_sc as plsc

N = 10000
E = 320000
D = 128

NUM_CORES = 2
NUM_TILES = 16
CHUNK = 128
NP = 10240
HP = NP // 2
CH_PER_TILE = 160
E_PAD = NUM_TILES * CH_PER_TILE * CHUNK
CH_ST = 16
N_STAGES = CH_PER_TILE // CH_ST


def _sc_aggregate(hp0, hp1, gi2, sp2, si2, qp2, ew2, zeros):
    mesh = plsc.VectorSubcoreMesh(core_axis_name="c", subcore_axis_name="s")

    @functools.partial(
        pl.kernel,
        mesh=mesh,
        out_type=jax.ShapeDtypeStruct((NP, D), jnp.float32),
        scratch_types=[
            pltpu.VMEM((CH_ST, CHUNK), jnp.int32),
            pltpu.VMEM((CH_ST, CHUNK), jnp.int32),
            pltpu.VMEM((CH_ST, CHUNK), jnp.int32),
            pltpu.VMEM((CH_ST, CHUNK), jnp.int32),
            pltpu.VMEM((CH_ST, CHUNK), jnp.float32),
            pltpu.VMEM((CHUNK, D), jnp.float32),
            pltpu.VMEM((CHUNK, D), jnp.float32),
            pltpu.VMEM_SHARED((HP, D), jnp.float32),
            pltpu.VMEM_SHARED((HP, D), jnp.float32),
            pltpu.SemaphoreType.DMA,
            pltpu.SemaphoreType.DMA,
            pltpu.SemaphoreType.DMA,
            pltpu.SemaphoreType.DMA,
        ],
    )
    def k(hp0_hbm, hp1_hbm, gi_hbm, sp_hbm, si_hbm, qp_hbm, w_hbm, z_hbm,
          out_hbm, gi_v, sp_v, si_v, qp_v, w_v, ba, bb, repl, acc,
          sem_ga, sem_gb, sem_sa, sem_sb):
        cid = lax.axis_index("c")
        sid = lax.axis_index("s")

        hrows = HP // NUM_TILES
        rsl = pl.ds(sid * hrows, hrows)

        @pl.when(cid == 0)
        def _():
            pltpu.sync_copy(hp0_hbm.at[rsl], repl.at[rsl])

        @pl.when(cid == 1)
        def _():
            pltpu.sync_copy(hp1_hbm.at[rsl], repl.at[rsl])

        pltpu.sync_copy(z_hbm.at[pl.ds(0, hrows)], acc.at[rsl])
        plsc.subcore_barrier()

        def process(buf, i):
            def group_body(g, c2):
                sl16 = pl.ds(g * 16, 16)
                w16 = w_v[i, sl16]
                sp16 = sp_v[i, sl16]
                qp16 = qp_v[i, sl16]
                zz = jnp.zeros((16,), jnp.float32)
                for j in range(16):
                    ws = jnp.full((16,), w16[j], jnp.float32)
                    po = sp16[j]
                    qo = qp16[j]
                    r = g * 16 + j
                    vals = [buf[r, pl.ds(po + 16 * kk, 16)]
                            for kk in range(4)]
                    for kk in range(4):
                        buf[r, pl.ds(qo + 16 * kk, 16)] = vals[kk] * ws
                    for kk in range(4):
                        buf[r, pl.ds((64 - qo) + 16 * kk, 16)] = zz
                return c2
            lax.fori_loop(0, CHUNK // 16, group_body, 0)

        def stage_body(st, carry):
            row0 = sid * CH_PER_TILE + st * CH_ST
            ssl = pl.ds(row0, CH_ST)
            pltpu.sync_copy(gi_hbm.at[ssl], gi_v)
            pltpu.sync_copy(sp_hbm.at[ssl], sp_v)
            pltpu.sync_copy(si_hbm.at[ssl], si_v)
            pltpu.sync_copy(qp_hbm.at[ssl], qp_v)
            pltpu.sync_copy(w_hbm.at[ssl], w_v)

            pltpu.async_copy(repl.at[gi_v.at[0]], ba, sem_ga)
            pltpu.async_copy(z_hbm.at[pl.ds(0, CHUNK)], bb, sem_sb)

            def pair_body(jj, c2):
                i0 = 2 * jj
                i1 = i0 + 1
                pltpu.make_async_copy(repl.at[gi_v.at[i0]], ba, sem_ga).wait()
                pltpu.make_async_copy(bb, acc.at[si_v.at[i1]], sem_sb).wait()
                pltpu.async_copy(repl.at[gi_v.at[i1]], bb, sem_gb)
                process(ba, i0)
                pltpu.async_copy(ba, acc.at[si_v.at[i0]], sem_sa, add=True)
                i2 = jnp.minimum(i0 + 2, CH_ST - 1)
                pltpu.make_async_copy(repl.at[gi_v.at[i1]], bb, sem_gb).wait()
                pltpu.make_async_copy(ba, acc.at[si_v.at[i0]], sem_sa).wait()
                pltpu.async_copy(repl.at[gi_v.at[i2]], ba, sem_ga)
                process(bb, i1)
                pltpu.async_copy(bb, acc.at[si_v.at[i1]], sem_sb, add=True)
                return c2

            lax.fori_loop(0, CH_ST // 2, pair_body, 0)
            pltpu.make_async_copy(
                repl.at[gi_v.at[CH_ST - 1]], ba, sem_ga).wait()
            pltpu.make_async_copy(
                bb, acc.at[si_v.at[CH_ST - 1]], sem_sb).wait()
            return carry

        lax.fori_loop(0, N_STAGES, stage_body, 0)
        plsc.subcore_barrier()

        pltpu.sync_copy(acc.at[rsl],
                        out_hbm.at[pl.ds(cid * HP + sid * hrows, hrows)])

    return k(hp0, hp1, gi2, sp2, si2, qp2, ew2, zeros)


def _ln_block(x, w, b):
    m = jnp.mean(x, axis=-1, keepdims=True)
    xc = x - m
    v = jnp.mean(xc * xc, axis=-1, keepdims=True)
    return xc * lax.rsqrt(v + 1e-5) * w + b


ROW_BLK = 1000


def _tc_layer0_body(a_ref, p_ref, h_ref, wrel_ref, wroot_ref,
                    brel_ref, lnw_ref, lnb_ref, o_ref):
    x = (jnp.dot(p_ref[...], wrel_ref[...], preferred_element_type=jnp.float32)
         + jnp.dot(h_ref[...], wroot_ref[...], preferred_element_type=jnp.float32)
         + brel_ref[...])
    y = _ln_block(x, lnw_ref[...], lnb_ref[...])
    a = a_ref[0]
    o_ref[...] = jnp.where(y >= 0, y, a * y)


def _tc_layer1_head_body(a_ref, p_ref, h_ref, wrel_ref, wroot_ref,
                         brel_ref, lnw_ref, lnb_ref, wc1_ref, bc1_ref,
                         lnwc_ref, lnbc_ref, wc2_ref, bc2_ref, o_ref):
    x = (jnp.dot(p_ref[...], wrel_ref[...], preferred_element_type=jnp.float32)
         + jnp.dot(h_ref[...], wroot_ref[...], preferred_element_type=jnp.float32)
         + brel_ref[...])
    y = _ln_block(x, lnw_ref[...], lnb_ref[...])
    a = a_ref[0]
    h2 = jnp.where(y >= 0, y, a * y)
    h3 = jnp.maximum(
        jnp.dot(h2, wc1_ref[...], preferred_element_type=jnp.float32)
        + bc1_ref[...], 0.0)
    h4 = _ln_block(h3, lnwc_ref[...], lnbc_ref[...])
    o_ref[...] = (jnp.dot(h4, wc2_ref[...], preferred_element_type=jnp.float32)
                  + bc2_ref[...])


def _row_spec():
    return pl.BlockSpec((ROW_BLK, D), lambda i: (i, 0))


def _full_spec():
    return pl.BlockSpec((D, D), lambda i: (0, 0))


def _vec_spec():
    return pl.BlockSpec((1, D), lambda i: (0, 0))


def _tc_layer0(p, h, wrel, wroot, brel, lnw, lnb, a):
    grid = (N // ROW_BLK,)
    return pl.pallas_call(
        _tc_layer0_body,
        grid=grid,
        in_specs=[
            pl.BlockSpec(memory_space=pltpu.SMEM),
            _row_spec(), _row_spec(),
            _full_spec(), _full_spec(),
            _vec_spec(), _vec_spec(), _vec_spec(),
        ],
        out_specs=_row_spec(),
        out_shape=jax.ShapeDtypeStruct((N, D), jnp.float32),
    )(a.reshape(1), p, h, wrel, wroot,
      brel.reshape(1, D), lnw.reshape(1, D), lnb.reshape(1, D))


def _tc_layer1_head(p, h, wrel, wroot, brel, lnw, lnb, a,
                    wc1, bc1, lnwc, lnbc, wc2p, bc2p):
    grid = (N // ROW_BLK,)
    return pl.pallas_call(
        _tc_layer1_head_body,
        grid=grid,
        in_specs=[
            pl.BlockSpec(memory_space=pltpu.SMEM),
            _row_spec(), _row_spec(),
            _full_spec(), _full_spec(),
            _vec_spec(), _vec_spec(), _vec_spec(),
            _full_spec(), _vec_spec(), _vec_spec(), _vec_spec(),
            _full_spec(), _vec_spec(),
        ],
        out_specs=_row_spec(),
        out_shape=jax.ShapeDtypeStruct((N, D), jnp.float32),
    )(a.reshape(1), p, h, wrel, wroot,
      brel.reshape(1, D), lnw.reshape(1, D), lnb.reshape(1, D),
      wc1, bc1.reshape(1, D), lnwc.reshape(1, D), lnbc.reshape(1, D),
      wc2p, bc2p.reshape(1, D))


def kernel(features, edge_index, edgenet_input, W_rel0, b_rel0, W_root0,
           ln_w0, ln_b0, prelu_a0, W_rel1, b_rel1, W_root1, ln_w1, ln_b1,
           prelu_a1, W_c1, b_c1, ln_wc, ln_bc, W_c2, b_c2):
    pad = E_PAD - E
    src = jnp.pad(edge_index[0], (0, pad))
    dst = jnp.pad(edge_index[1], (0, pad))
    sh2 = (E_PAD // CHUNK, CHUNK)
    gi2 = (src // 2).reshape(sh2)
    sp2 = ((src & 1) * 64).reshape(sh2)
    si2 = (dst // 2).reshape(sh2)
    qp2 = ((dst & 1) * 64).reshape(sh2)
    ew2 = jnp.pad(edgenet_input.reshape(-1), (0, pad)).reshape(sh2)
    zeros = jnp.zeros((HP // NUM_TILES, D), jnp.float32)

    def pack_halves(h):
        hpad = jnp.pad(h, ((0, NP - N), (0, 0)))
        return (hpad[:, :64].reshape(HP, D), hpad[:, 64:].reshape(HP, D))

    def unpack_aggr(res):
        return jnp.concatenate(
            [res[:HP].reshape(NP, 64), res[HP:].reshape(NP, 64)], axis=1)[:N]

    hp0, hp1 = pack_halves(features)
    aggr0 = unpack_aggr(_sc_aggregate(hp0, hp1, gi2, sp2, si2, qp2, ew2,
                                      zeros))
    h1 = _tc_layer0(aggr0, features,
                    W_rel0, W_root0, b_rel0, ln_w0, ln_b0,
                    jnp.asarray(prelu_a0, jnp.float32))

    hq0, hq1 = pack_halves(h1)
    aggr1 = unpack_aggr(_sc_aggregate(hq0, hq1, gi2, sp2, si2, qp2, ew2,
                                      zeros))
    wc2p = jnp.pad(W_c2, ((0, 0), (0, D - W_c2.shape[1])))
    bc2p = jnp.pad(b_c2, (0, D - b_c2.shape[0]))
    out = _tc_layer1_head(aggr1, h1,
                          W_rel1, W_root1, b_rel1, ln_w1, ln_b1,
                          jnp.asarray(prelu_a1, jnp.float32),
                          W_c1, b_c1, ln_wc, ln_bc, wc2p, bc2p)
    return out[:, :2]


# speedup vs baseline: 1.6063x; 1.6063x over previous
import functools

import jax
import jax.numpy as jnp
from jax import lax
from jax.experimental import pallas as pl
from jax.experimental.pallas import tpu as pltpu
from jax.experimental.pallas import tpu_sc as plsc

N = 10000
E = 320000
D = 128

NUM_CORES = 2
NUM_TILES = 16
NUM_WORKERS = NUM_CORES * NUM_TILES
E_PER_TILE = E // NUM_WORKERS
CHUNK = 80
N_CHUNKS = E_PER_TILE // CHUNK
NP = 10240
ROWS_PER_TILE = NP // NUM_TILES


def _sc_aggregate(h, src, dst, ew, zeros):
    mesh = plsc.VectorSubcoreMesh(core_axis_name="c", subcore_axis_name="s")

    @functools.partial(
        pl.kernel,
        mesh=mesh,
        out_type=jax.ShapeDtypeStruct((2 * NP, D), jnp.float32),
        scratch_types=[
            pltpu.VMEM((CHUNK,), jnp.int32),
            pltpu.VMEM((CHUNK,), jnp.int32),
            pltpu.VMEM((CHUNK,), jnp.float32),
            pltpu.VMEM((CHUNK, D), jnp.float32),
            pltpu.VMEM_SHARED((NP, D), jnp.float32),
            pltpu.SemaphoreType.DMA,
        ],
    )
    def k(h_hbm, src_hbm, dst_hbm, w_hbm, z_hbm, out_hbm,
          src_v, dst_v, w_v, rows_v, acc_sh, sem):
        cid = lax.axis_index("c")
        sid = lax.axis_index("s")

        pltpu.sync_copy(z_hbm.at[pl.ds(sid * ROWS_PER_TILE, ROWS_PER_TILE)],
                        acc_sh.at[pl.ds(sid * ROWS_PER_TILE, ROWS_PER_TILE)])
        plsc.subcore_barrier()

        wid = sid * NUM_CORES + cid
        base = wid * E_PER_TILE

        def chunk_body(i, carry):
            off = base + i * CHUNK
            pltpu.sync_copy(src_hbm.at[pl.ds(off, CHUNK)], src_v)
            pltpu.sync_copy(dst_hbm.at[pl.ds(off, CHUNK)], dst_v)
            pltpu.sync_copy(w_hbm.at[pl.ds(off, CHUNK)], w_v)
            pltpu.async_copy(h_hbm.at[src_v], rows_v, sem).wait()

            def group_body(g, c2):
                wg = w_v[pl.ds(g * 16, 16)]
                for j in range(16):
                    w16 = jnp.full((16,), wg[j], jnp.float32)
                    r = g * 16 + j
                    for kk in range(D // 16):
                        sl = pl.ds(kk * 16, 16)
                        rows_v[r, sl] = rows_v[r, sl] * w16
                return c2
            lax.fori_loop(0, CHUNK // 16, group_body, 0)

            pltpu.sync_copy(rows_v, acc_sh.at[dst_v], add=True)
            return carry

        lax.fori_loop(0, N_CHUNKS, chunk_body, 0)
        plsc.subcore_barrier()

        pltpu.sync_copy(
            acc_sh.at[pl.ds(sid * ROWS_PER_TILE, ROWS_PER_TILE)],
            out_hbm.at[pl.ds(cid * NP + sid * ROWS_PER_TILE, ROWS_PER_TILE)])

    return k(h, src, dst, ew, zeros)


def _ln_block(x, w, b):
    m = jnp.mean(x, axis=-1, keepdims=True)
    xc = x - m
    v = jnp.mean(xc * xc, axis=-1, keepdims=True)
    return xc * lax.rsqrt(v + 1e-5) * w + b


ROW_BLK = 1000


def _tc_layer0_body(a_ref, p0_ref, p1_ref, h_ref, wrel_ref, wroot_ref,
                    brel_ref, lnw_ref, lnb_ref, o_ref):
    aggr = p0_ref[...] + p1_ref[...]
    x = (jnp.dot(aggr, wrel_ref[...], preferred_element_type=jnp.float32)
         + jnp.dot(h_ref[...], wroot_ref[...], preferred_element_type=jnp.float32)
         + brel_ref[...])
    y = _ln_block(x, lnw_ref[...], lnb_ref[...])
    a = a_ref[0]
    o_ref[...] = jnp.where(y >= 0, y, a * y)


def _tc_layer1_head_body(a_ref, p0_ref, p1_ref, h_ref, wrel_ref, wroot_ref,
                         brel_ref, lnw_ref, lnb_ref, wc1_ref, bc1_ref,
                         lnwc_ref, lnbc_ref, wc2_ref, bc2_ref, o_ref):
    aggr = p0_ref[...] + p1_ref[...]
    x = (jnp.dot(aggr, wrel_ref[...], preferred_element_type=jnp.float32)
         + jnp.dot(h_ref[...], wroot_ref[...], preferred_element_type=jnp.float32)
         + brel_ref[...])
    y = _ln_block(x, lnw_ref[...], lnb_ref[...])
    a = a_ref[0]
    h2 = jnp.where(y >= 0, y, a * y)
    h3 = jnp.maximum(
        jnp.dot(h2, wc1_ref[...], preferred_element_type=jnp.float32)
        + bc1_ref[...], 0.0)
    h4 = _ln_block(h3, lnwc_ref[...], lnbc_ref[...])
    o_ref[...] = (jnp.dot(h4, wc2_ref[...], preferred_element_type=jnp.float32)
                  + bc2_ref[...])


def _row_spec():
    return pl.BlockSpec((ROW_BLK, D), lambda i: (i, 0))


def _full_spec():
    return pl.BlockSpec((D, D), lambda i: (0, 0))


def _vec_spec():
    return pl.BlockSpec((1, D), lambda i: (0, 0))


def _tc_layer0(p0, p1, h, wrel, wroot, brel, lnw, lnb, a):
    grid = (N // ROW_BLK,)
    return pl.pallas_call(
        _tc_layer0_body,
        grid=grid,
        in_specs=[
            pl.BlockSpec(memory_space=pltpu.SMEM),
            _row_spec(), _row_spec(), _row_spec(),
            _full_spec(), _full_spec(),
            _vec_spec(), _vec_spec(), _vec_spec(),
        ],
        out_specs=_row_spec(),
        out_shape=jax.ShapeDtypeStruct((N, D), jnp.float32),
    )(a.reshape(1), p0, p1, h, wrel, wroot,
      brel.reshape(1, D), lnw.reshape(1, D), lnb.reshape(1, D))


def _tc_layer1_head(p0, p1, h, wrel, wroot, brel, lnw, lnb, a,
                    wc1, bc1, lnwc, lnbc, wc2p, bc2p):
    grid = (N // ROW_BLK,)
    return pl.pallas_call(
        _tc_layer1_head_body,
        grid=grid,
        in_specs=[
            pl.BlockSpec(memory_space=pltpu.SMEM),
            _row_spec(), _row_spec(), _row_spec(),
            _full_spec(), _full_spec(),
            _vec_spec(), _vec_spec(), _vec_spec(),
            _full_spec(), _vec_spec(), _vec_spec(), _vec_spec(),
            _full_spec(), _vec_spec(),
        ],
        out_specs=_row_spec(),
        out_shape=jax.ShapeDtypeStruct((N, D), jnp.float32),
    )(a.reshape(1), p0, p1, h, wrel, wroot,
      brel.reshape(1, D), lnw.reshape(1, D), lnb.reshape(1, D),
      wc1, bc1.reshape(1, D), lnwc.reshape(1, D), lnbc.reshape(1, D),
      wc2p, bc2p.reshape(1, D))


def kernel(features, edge_index, edgenet_input, W_rel0, b_rel0, W_root0,
           ln_w0, ln_b0, prelu_a0, W_rel1, b_rel1, W_root1, ln_w1, ln_b1,
           prelu_a1, W_c1, b_c1, ln_wc, ln_bc, W_c2, b_c2):
    src = edge_index[0]
    dst = edge_index[1]
    ew = edgenet_input.reshape(-1)
    zeros = jnp.zeros((NP, D), jnp.float32)

    parts0 = _sc_aggregate(features, src, dst, ew, zeros)
    h1 = _tc_layer0(parts0[:N], parts0[NP:NP + N], features,
                    W_rel0, W_root0, b_rel0, ln_w0, ln_b0,
                    jnp.asarray(prelu_a0, jnp.float32))

    parts1 = _sc_aggregate(h1, src, dst, ew, zeros)
    wc2p = jnp.pad(W_c2, ((0, 0), (0, D - W_c2.shape[1])))
    bc2p = jnp.pad(b_c2, (0, D - b_c2.shape[0]))
    out = _tc_layer1_head(parts1[:N], parts1[NP:NP + N], h1,
                          W_rel1, W_root1, b_rel1, ln_w1, ln_b1,
                          jnp.asarray(prelu_a1, jnp.float32),
                          W_c1, b_c1, ln_wc, ln_bc, wc2p, bc2p)
    return out[:, :2]
